# baseline (speedup 1.0000x reference)
import functools

import jax
import jax.numpy as jnp
from jax import lax
from jax.experimental import pallas as pl
from jax.experimental.pallas import tpu as pltpu
from jax.experimental.pallas import tpu_sc as plsc

B = 16384
EMB = 64
NF = 26
FEMB = 32
FEAT_VOCAB = 1000
NUM_USERS = 1000000
NUM_ITEMS = 100000

NW = 32
PER_W = B // NW
C = 16
CHUNKS = PER_W // C
CF = C * NF
IDX_W = 104
NDMA = CF // IDX_W
FIW = CF * FEMB

_mesh = plsc.VectorSubcoreMesh(core_axis_name="c", subcore_axis_name="s")

_BUF = [
    pltpu.VMEM((C,), jnp.int32),
    pltpu.VMEM((C,), jnp.int32),
    pltpu.VMEM((CF,), jnp.int32),
    pltpu.VMEM((CF,), jnp.int32),
    pltpu.VMEM((FIW,), jnp.int32),
    pltpu.VMEM((C * EMB,), jnp.int32),
    pltpu.VMEM((C * EMB,), jnp.int32),
    pltpu.VMEM((C * EMB,), jnp.float32),
    pltpu.VMEM((C * EMB,), jnp.float32),
    pltpu.VMEM((C,), jnp.float32),
    pltpu.VMEM((C,), jnp.float32),
    pltpu.VMEM((CF, FEMB), jnp.float32),
    pltpu.VMEM((FIW,), jnp.float32),
    pltpu.SemaphoreType.DMA,
    pltpu.SemaphoreType.DMA,
]


@functools.partial(
    pl.kernel,
    out_type=jax.ShapeDtypeStruct((B,), jnp.float32),
    mesh=_mesh,
    compiler_params=pltpu.CompilerParams(use_tc_tiling_on_sc=False),
    scratch_types=_BUF + _BUF + [
        pltpu.VMEM((C,), jnp.float32),
        pltpu.VMEM((256,), jnp.float32),
        pltpu.VMEM((16,), jnp.float32),
        pltpu.VMEM((48,), jnp.int32),
        pltpu.VMEM((256,), jnp.int32),
    ],
)
def _mf_sc(u_id, i_id, feats, user_emb, user_bias, item_emb, item_bias,
           fu_tab, fi_tab, mean, out_hbm, *refs):
    bufs = (refs[:len(_BUF)], refs[len(_BUF):2 * len(_BUF)])
    out_v, P_v, mean_v, T_v, S_v = refs[2 * len(_BUF):]
    fi_flat = fi_tab.at[0]
    u_flat = user_emb.at[0]
    i_flat = item_emb.at[0]

    wid = lax.axis_index("s") * 2 + lax.axis_index("c")
    base0 = wid * PER_W

    pltpu.sync_copy(mean, mean_v)
    lanes = lax.iota(jnp.int32, 16)

    def idx_copies(t, d):
        u_v, i_v, f_v = bufs[d][0], bufs[d][1], bufs[d][2]
        isem = bufs[d][13]
        base = base0 + t * C
        return [
            pltpu.make_async_copy(u_id.at[pl.ds(base, C)], u_v, isem),
            pltpu.make_async_copy(i_id.at[pl.ds(base, C)], i_v, isem),
            pltpu.make_async_copy(feats.at[pl.ds(base * NF, CF)], f_v, isem),
        ]

    def gather_copies(d):
        (u_v, i_v, _, fu_idx, fi_idx, u_idx, i_idx, U_d, I_d,
         bu_v, bi_v, FU_v, FI_v, _, gsem) = bufs[d]
        cps = [
            pltpu.make_async_copy(user_bias.at[u_v], bu_v, gsem),
            pltpu.make_async_copy(item_bias.at[i_v], bi_v, gsem),
        ]
        for j in range(C * EMB // 128):
            s = pl.ds(j * 128, 128)
            cps.append(pltpu.make_async_copy(u_flat.at[u_idx.at[s]],
                                             U_d.at[s], gsem))
            cps.append(pltpu.make_async_copy(i_flat.at[i_idx.at[s]],
                                             I_d.at[s], gsem))
        for j in range(NDMA):
            s = pl.ds(j * IDX_W, IDX_W)
            cps.append(pltpu.make_async_copy(fu_tab.at[fu_idx.at[s]],
                                             FU_v.at[s], gsem))
        for j in range(FIW // 128):
            s = pl.ds(j * 128, 128)
            cps.append(pltpu.make_async_copy(fi_flat.at[fi_idx.at[s]],
                                             FI_v.at[s], gsem))
        return cps

    def build_and_fire(d):
        (u_v, i_v, f_v, fu_idx, fi_idx, u_idx, i_idx, *_) = bufs[d]
        uvec = u_v[...]
        ivec = i_v[...]
        for j in range(EMB):
            s = pl.ds(j * C, 16)
            u_idx[s] = uvec + j * NUM_USERS
            i_idx[s] = ivec + j * NUM_ITEMS
        for j in range(CF // 16):
            s = pl.ds(j * 16, 16)
            fu_idx[s] = f_v[s] + j * FEAT_VOCAB

        rows = [i_v[...]]
        for s in (8, 4, 2, 1):
            nxt = []
            for x in rows:
                T_v[pl.ds(s, 16)] = x
                b0 = T_v[pl.ds(0, 16)]
                bs = T_v[pl.ds(s, 16)]
                b2 = T_v[pl.ds(2 * s, 16)]
                m = (lanes & s) == 0
                nxt.append(jnp.where(m, bs, b0))
                nxt.append(jnp.where(m, b2, bs))
            rows = nxt
        for b in range(16):
            S_v[pl.ds(b * 16, 16)] = rows[b]

        lanes_n = lanes * NUM_ITEMS

        def fi_row(r, c2):
            ib = S_v[pl.ds((r & 15) * 16, 16)]
            sb = lax.shift_right_logical(r, 4) * (FEMB * NUM_ITEMS)
            fi_idx[pl.ds(r * FEMB, 16)] = ib + (lanes_n + sb)
            fi_idx[pl.ds(r * FEMB + 16, 16)] = ib + (
                lanes_n + (sb + 16 * NUM_ITEMS))
            return c2

        lax.fori_loop(0, CF, fi_row, 0, unroll=False)
        for cp in gather_copies(d):
            cp.start()

    def compute(t, d):
        (_, _, _, _, _, _, _, U_d, I_d, bu_v, bi_v, FU_v, FI_v,
         *_) = bufs[d]

        acc_ui = U_d[pl.ds(0, 16)] * I_d[pl.ds(0, 16)]
        for j in range(1, EMB):
            s = pl.ds(j * C, 16)
            acc_ui = acc_ui + U_d[s] * I_d[s]

        def elt(b, c2):
            acc = jnp.zeros((16,), jnp.float32)
            for f in range(NF):
                r = f * C + b
                for h in range(FEMB // 16):
                    acc = acc + (FU_v[r, pl.ds(h * 16, 16)]
                                 * FI_v[pl.ds(r * FEMB + h * 16, 16)])
            br = ((b & 1) << 3) | ((b & 2) << 1) | ((b & 4) >> 1) | ((b & 8) >> 3)
            P_v[pl.ds(br * 16, 16)] = acc
            return c2

        lax.fori_loop(0, C, elt, 0, unroll=False)

        for rnd, (w, n) in enumerate([(8, 8), (4, 4), (2, 2), (1, 1)]):
            for k in range(n):
                a0 = 32 * k
                t1 = P_v[pl.ds(a0, 16)] + P_v[pl.ds(a0 + w, 16)]
                t2 = P_v[pl.ds(a0 + 16 - w, 16)] + P_v[pl.ds(a0 + 16, 16)]
                sel = (lanes & (2 * w - 1)) < w
                q_ = jnp.where(sel, t1, t2)
                if rnd < 3:
                    P_v[pl.ds(16 * k, 16)] = q_

        s16 = pl.ds(0, 16)
        out_v[s16] = q_ + acc_ui + bu_v[s16] + bi_v[s16] + mean_v[s16]
        pltpu.sync_copy(out_v, out_hbm.at[pl.ds(base0 + t * C, C)])

    for cp in idx_copies(0, 0):
        cp.start()

    def step(tt, carry):
        for d in range(2):
            t = tt * 2 + d
            dn = 1 - d

            @pl.when((t > 0) & (t <= CHUNKS))
            def _():
                for cp in gather_copies(dn):
                    cp.wait()

            @pl.when(t + 1 < CHUNKS)
            def _():
                for cp in idx_copies(t + 1, dn):
                    cp.start()

            @pl.when(t < CHUNKS)
            def _():
                for cp in idx_copies(t, d):
                    cp.wait()
                build_and_fire(d)

            @pl.when((t > 0) & (t <= CHUNKS))
            def _():
                compute(t - 1, dn)

        return carry

    lax.fori_loop(0, CHUNKS // 2 + 1, step, 0, unroll=False)


def kernel(u_id, i_id, features, user_emb, user_bias, item_emb, item_bias,
           feat_u, feat_i, mean):
    u_id = u_id.astype(jnp.int32)
    i_id = i_id.astype(jnp.int32)
    feats = (features.astype(jnp.int32)
             .reshape(B // C, C, NF).transpose(0, 2, 1).reshape(-1))
    ue = user_emb.T.reshape(1, EMB * NUM_USERS)
    ie = item_emb.T.reshape(1, EMB * NUM_ITEMS)
    fu_tab = feat_u.reshape(NF * FEAT_VOCAB, FEMB)
    fi_tab = feat_i.transpose(0, 2, 1).reshape(1, NF * FEMB * NUM_ITEMS)
    ub = user_bias.reshape(-1)
    ib = item_bias.reshape(-1)
    mean16 = jnp.broadcast_to(mean, (16,))
    return _mf_sc(u_id, i_id, feats, ue, ub, ie,
                  ib, fu_tab, fi_tab, mean16)


# speedup vs baseline: 3.6757x; 3.6757x over previous
import functools

import jax
import jax.numpy as jnp
from jax import lax
from jax.experimental import pallas as pl
from jax.experimental.pallas import tpu as pltpu
from jax.experimental.pallas import tpu_sc as plsc

B = 16384
EMB = 64
NF = 26
FEMB = 32
FEAT_VOCAB = 1000
NUM_ITEMS = 100000

NW = 32
PER_W = B // NW
C = 16
CHUNKS = PER_W // C
CF = C * NF
IDX_W = 104
NDMA = CF // IDX_W
FIW = CF * FEMB

_mesh = plsc.VectorSubcoreMesh(core_axis_name="c", subcore_axis_name="s")

_BUF = [
    pltpu.VMEM((C,), jnp.int32),
    pltpu.VMEM((C,), jnp.int32),
    pltpu.VMEM((CF,), jnp.int32),
    pltpu.VMEM((CF,), jnp.int32),
    pltpu.VMEM((FIW,), jnp.int32),
    pltpu.VMEM((C, EMB), jnp.float32),
    pltpu.VMEM((C, EMB), jnp.float32),
    pltpu.VMEM((C,), jnp.float32),
    pltpu.VMEM((C,), jnp.float32),
    pltpu.VMEM((CF, FEMB), jnp.float32),
    pltpu.VMEM((FIW,), jnp.float32),
    pltpu.SemaphoreType.DMA,
    pltpu.SemaphoreType.DMA,
]


@functools.partial(
    pl.kernel,
    out_type=jax.ShapeDtypeStruct((B,), jnp.float32),
    mesh=_mesh,
    compiler_params=pltpu.CompilerParams(use_tc_tiling_on_sc=False),
    scratch_types=_BUF + _BUF + [
        pltpu.VMEM((C,), jnp.float32),
        pltpu.VMEM((256,), jnp.float32),
        pltpu.VMEM((16,), jnp.float32),
        pltpu.VMEM((48,), jnp.int32),
        pltpu.VMEM((256,), jnp.int32),
    ],
)
def _mf_sc(u_id, i_id, feats, user_emb, user_bias, item_emb, item_bias,
           fu_tab, fi_tab, mean, out_hbm, *refs):
    bufs = (refs[:len(_BUF)], refs[len(_BUF):2 * len(_BUF)])
    out_v, P_v, mean_v, T_v, S_v = refs[2 * len(_BUF):]
    fi_flat = fi_tab.at[0]

    wid = lax.axis_index("s") * 2 + lax.axis_index("c")
    base0 = wid * PER_W

    pltpu.sync_copy(mean, mean_v)
    lanes = lax.iota(jnp.int32, 16)

    def idx_copies(t, d):
        u_v, i_v, f_v = bufs[d][0], bufs[d][1], bufs[d][2]
        isem = bufs[d][11]
        base = base0 + t * C
        return [
            pltpu.make_async_copy(u_id.at[pl.ds(base, C)], u_v, isem),
            pltpu.make_async_copy(i_id.at[pl.ds(base, C)], i_v, isem),
            pltpu.make_async_copy(feats.at[pl.ds(base * NF, CF)], f_v, isem),
        ]

    def gather_copies(d):
        (u_v, i_v, _, fu_idx, fi_idx, U_v, I_v, bu_v, bi_v, FU_v, FI_v,
         _, gsem) = bufs[d]
        cps = [
            pltpu.make_async_copy(user_emb.at[u_v], U_v, gsem),
            pltpu.make_async_copy(item_emb.at[i_v], I_v, gsem),
            pltpu.make_async_copy(user_bias.at[u_v], bu_v, gsem),
            pltpu.make_async_copy(item_bias.at[i_v], bi_v, gsem),
        ]
        for j in range(NDMA):
            s = pl.ds(j * IDX_W, IDX_W)
            cps.append(pltpu.make_async_copy(fu_tab.at[fu_idx.at[s]],
                                             FU_v.at[s], gsem))
        for j in range(FIW // 128):
            s = pl.ds(j * 128, 128)
            cps.append(pltpu.make_async_copy(fi_flat.at[fi_idx.at[s]],
                                             FI_v.at[s], gsem))
        return cps

    def build_and_fire(d):
        (u_v, i_v, f_v, fu_idx, fi_idx, *_) = bufs[d]
        for j in range(CF // 16):
            s = pl.ds(j * 16, 16)
            fu_idx[s] = f_v[s] + j * FEAT_VOCAB

        rows = [i_v[...]]
        for s in (8, 4, 2, 1):
            nxt = []
            for x in rows:
                T_v[pl.ds(s, 16)] = x
                b0 = T_v[pl.ds(0, 16)]
                bs = T_v[pl.ds(s, 16)]
                b2 = T_v[pl.ds(2 * s, 16)]
                m = (lanes & s) == 0
                nxt.append(jnp.where(m, bs, b0))
                nxt.append(jnp.where(m, b2, bs))
            rows = nxt
        for b in range(16):
            S_v[pl.ds(b * 16, 16)] = rows[b]

        lanes_n = lanes * NUM_ITEMS

        def fi_row(r, c2):
            ib = S_v[pl.ds((r & 15) * 16, 16)]
            sb = lax.shift_right_logical(r, 4) * (FEMB * NUM_ITEMS)
            fi_idx[pl.ds(r * FEMB, 16)] = ib + (lanes_n + sb)
            fi_idx[pl.ds(r * FEMB + 16, 16)] = ib + (
                lanes_n + (sb + 16 * NUM_ITEMS))
            return c2

        lax.fori_loop(0, CF, fi_row, 0, unroll=False)
        for cp in gather_copies(d):
            cp.start()

    def compute(t, d):
        (_, _, _, _, _, U_v, I_v, bu_v, bi_v, FU_v, FI_v, *_) = bufs[d]

        def elt(b, c2):
            acc = U_v[b, pl.ds(0, 16)] * I_v[b, pl.ds(0, 16)]
            for k in range(1, EMB // 16):
                cs = pl.ds(k * 16, 16)
                acc = acc + U_v[b, cs] * I_v[b, cs]
            for f in range(NF):
                r = f * C + b
                for h in range(FEMB // 16):
                    acc = acc + (FU_v[r, pl.ds(h * 16, 16)]
                                 * FI_v[pl.ds(r * FEMB + h * 16, 16)])
            br = ((b & 1) << 3) | ((b & 2) << 1) | ((b & 4) >> 1) | ((b & 8) >> 3)
            P_v[pl.ds(br * 16, 16)] = acc
            return c2

        lax.fori_loop(0, C, elt, 0, unroll=False)

        for rnd, (w, n) in enumerate([(8, 8), (4, 4), (2, 2), (1, 1)]):
            for k in range(n):
                a0 = 32 * k
                t1 = P_v[pl.ds(a0, 16)] + P_v[pl.ds(a0 + w, 16)]
                t2 = P_v[pl.ds(a0 + 16 - w, 16)] + P_v[pl.ds(a0 + 16, 16)]
                sel = (lanes & (2 * w - 1)) < w
                q_ = jnp.where(sel, t1, t2)
                if rnd < 3:
                    P_v[pl.ds(16 * k, 16)] = q_

        s16 = pl.ds(0, 16)
        out_v[s16] = q_ + bu_v[s16] + bi_v[s16] + mean_v[s16]
        pltpu.sync_copy(out_v, out_hbm.at[pl.ds(base0 + t * C, C)])

    for cp in idx_copies(0, 0):
        cp.start()

    def step(tt, carry):
        for d in range(2):
            t = tt * 2 + d
            dn = 1 - d

            @pl.when((t > 0) & (t <= CHUNKS))
            def _():
                for cp in gather_copies(dn):
                    cp.wait()

            @pl.when(t + 1 < CHUNKS)
            def _():
                for cp in idx_copies(t + 1, dn):
                    cp.start()

            @pl.when(t < CHUNKS)
            def _():
                for cp in idx_copies(t, d):
                    cp.wait()
                build_and_fire(d)

            @pl.when((t > 0) & (t <= CHUNKS))
            def _():
                compute(t - 1, dn)

        return carry

    lax.fori_loop(0, CHUNKS // 2 + 1, step, 0, unroll=False)


def kernel(u_id, i_id, features, user_emb, user_bias, item_emb, item_bias,
           feat_u, feat_i, mean):
    u_id = u_id.astype(jnp.int32)
    i_id = i_id.astype(jnp.int32)
    feats = (features.astype(jnp.int32)
             .reshape(B // C, C, NF).transpose(0, 2, 1).reshape(-1))
    fu_tab = feat_u.reshape(NF * FEAT_VOCAB, FEMB)
    fi_tab = feat_i.transpose(0, 2, 1).reshape(1, NF * FEMB * NUM_ITEMS)
    ub = user_bias.reshape(-1)
    ib = item_bias.reshape(-1)
    mean16 = jnp.broadcast_to(mean, (16,))
    return _mf_sc(u_id, i_id, feats, user_emb, ub, item_emb,
                  ib, fu_tab, fi_tab, mean16)
